# baseline (speedup 1.0000x reference)
import jax
import jax.numpy as jnp
from jax.experimental import pallas as pl
from jax.experimental.pallas import tpu as pltpu


def _moe_body(x_ref, wg_ref, wgate_ref, wup_ref, wdown_ref, out_ref, wfull_ref):
    e = pl.program_id(0)
    f = pl.program_id(1)

    @pl.when((e == 0) & (f == 0))
    def _init():
        x = x_ref[...]
        logits = jax.lax.dot_general(
            x, wg_ref[...], (((1,), (1,)), ((), ())),
            preferred_element_type=jnp.float32)
        m = jnp.max(logits, axis=-1, keepdims=True)
        ex = jnp.exp(logits - m)
        p = ex / jnp.sum(ex, axis=-1, keepdims=True)
        num_e = p.shape[-1]
        col = jax.lax.broadcasted_iota(jnp.int32, p.shape, 1)
        m1 = jnp.max(p, axis=-1, keepdims=True)
        i1 = jnp.min(jnp.where(p == m1, col, num_e), axis=-1, keepdims=True)
        mask1 = col == i1
        p2 = jnp.where(mask1, -jnp.inf, p)
        m2 = jnp.max(p2, axis=-1, keepdims=True)
        i2 = jnp.min(jnp.where(p2 == m2, col, num_e), axis=-1, keepdims=True)
        mask2 = col == i2
        denom = m1 + m2
        wfull_ref[...] = (jnp.where(mask1, m1, 0.0)
                          + jnp.where(mask2, m2, 0.0)) / denom
        out_ref[...] = jnp.zeros_like(out_ref)

    x = x_ref[...]
    g = jnp.dot(x, wgate_ref[0], preferred_element_type=jnp.float32)
    u = jnp.dot(x, wup_ref[0], preferred_element_type=jnp.float32)
    h = g * jax.nn.sigmoid(g) * u
    o = jnp.dot(h, wdown_ref[0], preferred_element_type=jnp.float32)
    wfull = wfull_ref[...]
    ecol = jax.lax.broadcasted_iota(jnp.int32, wfull.shape, 1)
    w = jnp.sum(jnp.where(ecol == e, wfull, 0.0), axis=1, keepdims=True)
    out_ref[...] += w * o


def kernel(hidden_states, Wg, W_gate, W_up, W_down):
    T, D = hidden_states.shape
    E, _, F = W_gate.shape
    Fb = 128 if F % 128 == 0 else F
    NF = F // Fb
    return pl.pallas_call(
        _moe_body,
        grid=(E, NF),
        in_specs=[
            pl.BlockSpec((T, D), lambda e, f: (0, 0)),
            pl.BlockSpec((E, D), lambda e, f: (0, 0)),
            pl.BlockSpec((1, D, Fb), lambda e, f: (e, 0, f)),
            pl.BlockSpec((1, D, Fb), lambda e, f: (e, 0, f)),
            pl.BlockSpec((1, Fb, D), lambda e, f: (e, f, 0)),
        ],
        out_specs=pl.BlockSpec((T, D), lambda e, f: (0, 0)),
        out_shape=jax.ShapeDtypeStruct((T, D), jnp.float32),
        scratch_shapes=[pltpu.VMEM((T, E), jnp.float32)],
        compiler_params=pltpu.CompilerParams(
            dimension_semantics=("arbitrary", "arbitrary")),
    )(hidden_states, Wg, W_gate, W_up, W_down)
